# baseline (speedup 1.0000x reference)
import functools

import jax
import jax.numpy as jnp
from jax import lax
from jax.experimental import pallas as pl
from jax.experimental.pallas import tpu as pltpu
from jax.experimental.pallas import tpu_sc as plsc

BATCH = 16384
HIST = 50
WIDTH = 32
GRP = HIST + 1

NC = 2
NS = 16
NW = NC * NS
BPW = BATCH // NW
NB = 16
CROWS = NB * GRP
NCHUNK = BPW // NB
VPROWS = BPW // 128


def _sc_embed(cr51, vp2d, card_table, vp_table):
    mesh = plsc.VectorSubcoreMesh(core_axis_name="c", subcore_axis_name="s")

    @functools.partial(
        pl.kernel,
        mesh=mesh,
        compiler_params=pltpu.CompilerParams(use_tc_tiling_on_sc=False),
        out_type=jax.ShapeDtypeStruct((BATCH * GRP, WIDTH), jnp.float32),
        scratch_types=[
            pltpu.VMEM((2, CROWS), jnp.int32),
            pltpu.VMEM((VPROWS, 128), jnp.int32),
            pltpu.VMEM((2, NB, WIDTH), jnp.float32),
            pltpu.VMEM((2, CROWS, WIDTH), jnp.float32),
            pltpu.SemaphoreType.DMA,
            pltpu.SemaphoreType.DMA,
        ],
    )
    def k(cr_hbm, vp_hbm, ctab_hbm, vtab_hbm, out_hbm,
          idx_v, vpi_v, vpc_v, asm_v, gsem, wsem):
        wid = lax.axis_index("s") * NC + lax.axis_index("c")
        base = wid * BPW
        pltpu.sync_copy(vp_hbm.at[pl.ds(wid * VPROWS, VPROWS)], vpi_v)

        def fire(ci, slot):
            b0 = base + ci * NB
            pltpu.sync_copy(cr_hbm.at[pl.ds(b0 * GRP, CROWS)], idx_v.at[slot])
            r = ci // 8
            c0 = (ci % 8) * NB
            pltpu.async_copy(
                vtab_hbm.at[vpi_v.at[r, pl.ds(c0, NB)]], vpc_v.at[slot], gsem)
            pltpu.async_copy(
                ctab_hbm.at[idx_v.at[slot]], asm_v.at[slot], gsem)

        def drain(ci, slot):
            r = ci // 8
            c0 = (ci % 8) * NB
            pltpu.make_async_copy(
                vtab_hbm.at[vpi_v.at[r, pl.ds(c0, NB)]],
                vpc_v.at[slot], gsem).wait()
            pltpu.make_async_copy(
                ctab_hbm.at[idx_v.at[slot]], asm_v.at[slot], gsem).wait()

        fire(0, 0)

        def chunk(ci, carry):
            slot = ci % 2
            nslot = (ci + 1) % 2
            b0 = base + ci * NB
            drain(ci, slot)
            for i in range(NB):
                asm_v[slot, i * GRP + HIST, pl.ds(0, 16)] = (
                    vpc_v[slot, i, pl.ds(0, 16)])
                asm_v[slot, i * GRP + HIST, pl.ds(16, 16)] = (
                    vpc_v[slot, i, pl.ds(16, 16)])
            @pl.when(ci >= 1)
            def _():
                pltpu.make_async_copy(
                    asm_v.at[nslot], out_hbm.at[pl.ds(0, CROWS)], wsem).wait()
            pltpu.async_copy(
                asm_v.at[slot], out_hbm.at[pl.ds(b0 * GRP, CROWS)], wsem)

            @pl.when(ci + 1 < NCHUNK)
            def _():
                fire(ci + 1, nslot)
            return carry

        lax.fori_loop(0, NCHUNK, chunk, 0)
        pltpu.make_async_copy(
            asm_v.at[(NCHUNK - 1) % 2], out_hbm.at[pl.ds(0, CROWS)], wsem).wait()

    return k(cr51, vp2d, card_table, vp_table)


def kernel(observation, card_rewards, vp_rewards, cards, card_table, vp_table):
    del observation, cards
    cr = card_rewards.astype(jnp.int32)
    cr51 = jnp.concatenate(
        [cr, jnp.zeros((BATCH, 1), jnp.int32)], axis=1).reshape(-1)
    vp2d = vp_rewards.astype(jnp.int32).reshape(BATCH // 128, 128)
    out = _sc_embed(cr51, vp2d, card_table, vp_table)
    return out.reshape(BATCH, GRP, WIDTH)


# speedup vs baseline: 1.8433x; 1.8433x over previous
import functools

import jax
import jax.numpy as jnp
from jax import lax
from jax.experimental import pallas as pl
from jax.experimental.pallas import tpu as pltpu
from jax.experimental.pallas import tpu_sc as plsc

BATCH = 16384
HIST = 50
WIDTH = 32
GRP = HIST + 1

NC = 2
NS = 16
NW = NC * NS
BPW = BATCH // NW
NB = 16
NCHUNK = BPW // NB
VPROWS = BPW // 128

SPLITS = ((0, 24), (24, 26))


def _sc_embed(card_rewards, vp2d, card_table, vp_table):
    mesh = plsc.VectorSubcoreMesh(core_axis_name="c", subcore_axis_name="s")

    @functools.partial(
        pl.kernel,
        mesh=mesh,
        compiler_params=pltpu.CompilerParams(use_tc_tiling_on_sc=False),
        out_type=jax.ShapeDtypeStruct((BATCH, GRP, WIDTH), jnp.float32),
        scratch_types=[
            pltpu.VMEM((2, NB, HIST), jnp.int32),
            pltpu.VMEM((VPROWS, 128), jnp.int32),
            pltpu.VMEM((2, NB, WIDTH), jnp.float32),
            pltpu.VMEM((2, NB, GRP, WIDTH), jnp.float32),
            pltpu.SemaphoreType.DMA,
            pltpu.SemaphoreType.DMA,
        ],
    )
    def k(cr_hbm, vp_hbm, ctab_hbm, vtab_hbm, out_hbm,
          idx_v, vpi_v, vpc_v, asm_v, gsem, wsem):
        wid = lax.axis_index("s") * NC + lax.axis_index("c")
        base = wid * BPW
        pltpu.sync_copy(vp_hbm.at[pl.ds(wid * VPROWS, VPROWS)], vpi_v)

        def gathers(ci, slot, launch):
            op = pltpu.async_copy if launch else (
                lambda s, d, m: pltpu.make_async_copy(s, d, m).wait())
            r = ci // 8
            c0 = (ci % 8) * NB
            op(vtab_hbm.at[vpi_v.at[r, pl.ds(c0, NB)]], vpc_v.at[slot], gsem)
            for i in range(NB):
                for off, ln in SPLITS:
                    op(ctab_hbm.at[idx_v.at[slot, i, pl.ds(off, ln)]],
                       asm_v.at[slot, i, pl.ds(off, ln)], gsem)

        def fire(ci, slot):
            b0 = base + ci * NB
            pltpu.sync_copy(cr_hbm.at[pl.ds(b0, NB)], idx_v.at[slot])
            gathers(ci, slot, True)

        fire(0, 0)

        def chunk(ci, carry):
            slot = ci % 2
            nslot = (ci + 1) % 2
            b0 = base + ci * NB
            gathers(ci, slot, False)
            for i in range(NB):
                asm_v[slot, i, HIST, pl.ds(0, 16)] = vpc_v[slot, i, pl.ds(0, 16)]
                asm_v[slot, i, HIST, pl.ds(16, 16)] = vpc_v[slot, i, pl.ds(16, 16)]
            @pl.when(ci >= 1)
            def _():
                pltpu.make_async_copy(
                    asm_v.at[nslot], out_hbm.at[pl.ds(0, NB)], wsem).wait()
            pltpu.async_copy(asm_v.at[slot], out_hbm.at[pl.ds(b0, NB)], wsem)

            @pl.when(ci + 1 < NCHUNK)
            def _():
                fire(ci + 1, nslot)
            return carry

        lax.fori_loop(0, NCHUNK, chunk, 0)
        pltpu.make_async_copy(
            asm_v.at[(NCHUNK - 1) % 2], out_hbm.at[pl.ds(0, NB)], wsem).wait()

    return k(card_rewards, vp2d, card_table, vp_table)


def kernel(observation, card_rewards, vp_rewards, cards, card_table, vp_table):
    del observation, cards
    cr = card_rewards.astype(jnp.int32)
    vp2d = vp_rewards.astype(jnp.int32).reshape(BATCH // 128, 128)
    return _sc_embed(cr, vp2d, card_table, vp_table)
